# Initial kernel scaffold; baseline (speedup 1.0000x reference)
import jax
import jax.numpy as jnp
from jax.experimental import pallas as pl


def kernel(batch, table):
    raise NotImplementedError("write your pallas kernel here")


# speedup vs baseline: 4.8091x; 4.8091x over previous
import functools

import jax
import jax.numpy as jnp
from jax import lax
from jax.experimental import pallas as pl
from jax.experimental.pallas import tpu as pltpu
from jax.experimental.pallas import tpu_sc as plsc

_L = 128
_CH = 8


def _build(G, D, NC, NS):
    NW = NC * NS
    gpw = G // NW
    chunks = gpw // _CH
    mesh = plsc.VectorSubcoreMesh(core_axis_name="c", subcore_axis_name="s")

    @functools.partial(
        pl.kernel,
        mesh=mesh,
        out_type=jax.ShapeDtypeStruct((G, _L, D), jnp.float32),
        scratch_types=[
            pltpu.VMEM((_CH, _L), jnp.int32),
            pltpu.VMEM((_CH, _L, D), jnp.float32),
            pltpu.SemaphoreType.DMA,
        ],
        compiler_params=pltpu.CompilerParams(use_tc_tiling_on_sc=False),
    )
    def emb(table_hbm, idx_hbm, out_hbm, idx_v, rows_v, sem):
        wid = lax.axis_index("s") * NC + lax.axis_index("c")
        base = wid * gpw

        def chunk(i, carry):
            g0 = base + i * _CH
            pltpu.sync_copy(idx_hbm.at[pl.ds(g0, _CH)], idx_v)
            copies = [
                pltpu.async_copy(table_hbm.at[idx_v.at[j]], rows_v.at[j], sem)
                for j in range(_CH)
            ]
            for c in copies:
                c.wait()
            pltpu.sync_copy(rows_v, out_hbm.at[pl.ds(g0, _CH)])
            return carry

        lax.fori_loop(0, chunks, chunk, 0)

    return emb


def kernel(batch, table):
    B, H = batch.shape
    V, D = table.shape
    G = (B * H) // _L
    info = plsc.get_sparse_core_info()
    idx = batch.reshape(G, _L).astype(jnp.int32)
    out = _build(G, D, info.num_cores, info.num_subcores)(table, idx)
    return out.reshape(B, H, D)


# speedup vs baseline: 5.0464x; 1.0493x over previous
import functools

import jax
import jax.numpy as jnp
from jax import lax
from jax.experimental import pallas as pl
from jax.experimental.pallas import tpu as pltpu
from jax.experimental.pallas import tpu_sc as plsc

_L = 128
_CH = 8


def _build(G, D, NC, NS):
    NW = NC * NS
    gpw = G // NW
    chunks = gpw // _CH
    mesh = plsc.VectorSubcoreMesh(core_axis_name="c", subcore_axis_name="s")

    @functools.partial(
        pl.kernel,
        mesh=mesh,
        out_type=jax.ShapeDtypeStruct((G, _L, D), jnp.float32),
        scratch_types=[
            pltpu.VMEM((_CH, _L), jnp.int32),
            pltpu.VMEM((_CH, _L), jnp.int32),
            pltpu.VMEM((_CH, _L, D), jnp.float32),
            pltpu.VMEM((_CH, _L, D), jnp.float32),
            pltpu.SemaphoreType.DMA,
            pltpu.SemaphoreType.DMA,
            pltpu.SemaphoreType.DMA,
            pltpu.SemaphoreType.DMA,
            pltpu.SemaphoreType.DMA,
            pltpu.SemaphoreType.DMA,
        ],
        compiler_params=pltpu.CompilerParams(use_tc_tiling_on_sc=False),
    )
    def emb(table_hbm, idx_hbm, out_hbm, i0, i1, r0, r1,
            si0, si1, sg0, sg1, so0, so1):
        idx_v = (i0, i1)
        rows_v = (r0, r1)
        s_i = (si0, si1)
        s_g = (sg0, sg1)
        s_o = (so0, so1)
        wid = lax.axis_index("s") * NC + lax.axis_index("c")
        base = wid * gpw

        def idx_src(c):
            return idx_hbm.at[pl.ds(base + c * _CH, _CH)]

        def out_dst(c):
            return out_hbm.at[pl.ds(base + c * _CH, _CH)]

        def wait_idx(b):
            pltpu.make_async_copy(idx_src(0), idx_v[b], s_i[b]).wait()

        def wait_out(b):
            pltpu.make_async_copy(rows_v[b], out_dst(0), s_o[b]).wait()

        def wait_gathers(b):
            for j in range(_CH):
                pltpu.make_async_copy(
                    table_hbm.at[idx_v[b].at[j]], rows_v[b].at[j], s_g[b]
                ).wait()

        def fire_gathers(b):
            for j in range(_CH):
                pltpu.async_copy(
                    table_hbm.at[idx_v[b].at[j]], rows_v[b].at[j], s_g[b]
                )

        def stage_a(b, first):
            wait_idx(b)
            if not first:
                wait_out(b)
            fire_gathers(b)

        def stage_b(c, b, fetch_c):
            wait_gathers(b)
            if fetch_c is not None:
                pltpu.async_copy(idx_src(fetch_c), idx_v[b], s_i[b])
            pltpu.async_copy(rows_v[b], out_dst(c), s_o[b])

        pltpu.async_copy(idx_src(0), i0, si0)
        pltpu.async_copy(idx_src(1), i1, si1)
        stage_a(0, first=True)
        stage_a(1, first=True)
        stage_b(0, 0, fetch_c=2)

        last = chunks - 1

        def pair(p, carry):
            c0 = 2 * p
            stage_a(0, first=False)
            stage_b(c0 - 1, 1, jnp.minimum(c0 + 1, last))
            stage_a(1, first=False)
            stage_b(c0, 0, jnp.minimum(c0 + 2, last))
            return carry

        lax.fori_loop(1, chunks // 2, pair, 0)

        stage_b(last, 1, fetch_c=None)
        wait_idx(0)
        wait_out(0)
        wait_out(1)

    return emb


def kernel(batch, table):
    B, H = batch.shape
    V, D = table.shape
    G = (B * H) // _L
    info = plsc.get_sparse_core_info()
    idx = batch.reshape(G, _L).astype(jnp.int32)
    out = _build(G, D, info.num_cores, info.num_subcores)(table, idx)
    return out.reshape(B, H, D)
